# baseline (speedup 1.0000x reference)
import functools

import jax
import jax.numpy as jnp
from jax import lax
from jax.experimental import pallas as pl
from jax.experimental.pallas import tpu as pltpu
from jax.experimental.pallas import tpu_sc as plsc

_D = 64
_T = 200
_B = 4096
_NC = 2
_NS = 16
_NW = _NC * _NS
_BPW = _B // _NW
_L = 16
_NQ = _D // _L
_TCH = 2
_NCH = _T // _TCH
_PS = 129

_mesh = plsc.VectorSubcoreMesh(core_axis_name="c", subcore_axis_name="s")


@functools.partial(
    pl.kernel,
    out_type=jax.ShapeDtypeStruct((_T, _D // 8, _B // 128, 8, 128), jnp.float32),
    mesh=_mesh,
    compiler_params=pltpu.CompilerParams(
        use_tc_tiling_on_sc=False, needs_layout_passes=False),
    scratch_types=[
        pltpu.VMEM((_T, _BPW), jnp.int32),
        pltpu.VMEM((_BPW, _D), jnp.float32),
        pltpu.VMEM((2, _TCH * _BPW, _D), jnp.float32),
        pltpu.VMEM((2, _TCH, 8, 8, _PS), jnp.float32),
        pltpu.SemaphoreType.DMA,
        pltpu.SemaphoreType.DMA,
        pltpu.SemaphoreType.DMA,
        pltpu.SemaphoreType.DMA,
    ],
)
def _dummy_world_rnn(table_hbm, actT_hbm, z0_hbm, out_hbm,
                     idx_v, acc_v, rows_v, outP_v,
                     sem_g0, sem_g1, sem_w0, sem_w1):
    wid = lax.axis_index("s") * _NC + lax.axis_index("c")
    base = wid * _BPW
    sem_g = (sem_g0, sem_g1)
    sem_w = (sem_w0, sem_w1)

    pltpu.sync_copy(actT_hbm.at[:, pl.ds(base, _BPW)], idx_v)
    pltpu.sync_copy(z0_hbm.at[pl.ds(base, _BPW)], acc_v)

    iota = lax.iota(jnp.int32, _L)
    dtv = [jnp.int32(2 * q) + (iota >> 3) for q in range(_NQ)]
    drv = [iota & 7 for _ in range(_NQ)]

    def fire_gather(k, p):
        t0 = jnp.minimum(k, _NCH - 1) * _TCH
        for tl in range(_TCH):
            pltpu.async_copy(
                table_hbm.at[idx_v.at[t0 + tl, :]],
                rows_v.at[p, pl.ds(tl * _BPW, _BPW)], sem_g[p])

    def wait_gather(p):
        for _ in range(_TCH):
            pltpu.make_async_copy(
                table_hbm.at[pl.ds(0, _BPW)],
                rows_v.at[p, pl.ds(0, _BPW)], sem_g[p]).wait()

    def fire_write(k, p):
        t0 = k * _TCH
        for tl in range(_TCH):
            pltpu.async_copy(
                outP_v.at[p, tl, :, :, pl.ds(0, 128)],
                out_hbm.at[t0 + tl, :, wid], sem_w[p])

    def wait_write(p):
        for _ in range(_TCH):
            pltpu.make_async_copy(
                outP_v.at[p, 0, :, :, pl.ds(0, 128)],
                out_hbm.at[0, :, wid], sem_w[p]).wait()

    def compute(p):
        def bloop(b, carry):
            blv = jnp.full((_L,), b, jnp.int32)
            accs = [acc_v[b, pl.ds(q * _L, _L)] for q in range(_NQ)]
            for tl in range(_TCH):
                r = tl * _BPW + b
                for q in range(_NQ):
                    g = rows_v[p, r, pl.ds(q * _L, _L)]
                    accs[q] = accs[q] + g * 0.1
                    plsc.store_scatter(
                        outP_v.at[p, tl], [dtv[q], drv[q], blv], accs[q])
            for q in range(_NQ):
                acc_v[b, pl.ds(q * _L, _L)] = accs[q]
            return carry

        lax.fori_loop(0, _BPW, bloop, 0)

    def chunk(k, p, wait_out):
        wait_gather(p)
        if wait_out:
            wait_write(p)
        compute(p)
        fire_write(k, p)
        fire_gather(k + 2, p)

    fire_gather(0, 0)
    fire_gather(1, 1)
    chunk(0, 0, False)
    chunk(1, 1, False)

    def pair(i, carry):
        chunk(2 * i, 0, True)
        chunk(2 * i + 1, 1, True)
        return carry

    lax.fori_loop(1, _NCH // 2, pair, 0)

    wait_gather(0)
    wait_gather(1)
    wait_write(0)
    wait_write(1)


def kernel(z0, action_seq, act_emb_weight):
    out5 = _dummy_world_rnn(
        act_emb_weight,
        action_seq.astype(jnp.int32).T,
        z0,
    )
    return out5.transpose(2, 4, 0, 1, 3).reshape(_B, _T, _D)


# speedup vs baseline: 2.8722x; 2.8722x over previous
import functools

import jax
import jax.numpy as jnp
from jax import lax
from jax.experimental import pallas as pl
from jax.experimental.pallas import tpu as pltpu
from jax.experimental.pallas import tpu_sc as plsc

_D = 64
_T = 200
_B = 4096
_NC = 2
_NS = 16
_NW = _NC * _NS
_BPW = _B // _NW
_L = 16
_NBG = _BPW // _L
_TCH = 2
_NCH = _T // _TCH
_RS = _D + 1

_mesh = plsc.VectorSubcoreMesh(core_axis_name="c", subcore_axis_name="s")


@functools.partial(
    pl.kernel,
    out_type=jax.ShapeDtypeStruct((_T, _D // 8, _B // 128, 8, 128), jnp.float32),
    mesh=_mesh,
    compiler_params=pltpu.CompilerParams(
        use_tc_tiling_on_sc=False, needs_layout_passes=False),
    scratch_types=[
        pltpu.VMEM((_T, _BPW), jnp.int32),
        pltpu.VMEM((_D, _BPW), jnp.float32),
        pltpu.VMEM((2, _TCH * _BPW, _D), jnp.float32),
        pltpu.VMEM((_TCH * _BPW * _RS,), jnp.float32),
        pltpu.VMEM((2, _TCH, 8, 8, 128), jnp.float32),
        pltpu.SemaphoreType.DMA,
        pltpu.SemaphoreType.DMA,
        pltpu.SemaphoreType.DMA,
        pltpu.SemaphoreType.DMA,
    ],
)
def _dummy_world_rnn(table_hbm, actT_hbm, z0T_hbm, out_hbm,
                     idx_v, acc_v, rows_v, rowsP_v, out_v,
                     sem_g0, sem_g1, sem_w0, sem_w1):
    wid = lax.axis_index("s") * _NC + lax.axis_index("c")
    base = wid * _BPW
    sem_g = (sem_g0, sem_g1)
    sem_w = (sem_w0, sem_w1)

    pltpu.sync_copy(actT_hbm.at[:, pl.ds(base, _BPW)], idx_v)
    pltpu.sync_copy(z0T_hbm.at[:, pl.ds(base, _BPW)], acc_v)

    iota = lax.iota(jnp.int32, _L)
    gbase = [[(jnp.int32(tl * _BPW + bg * _L) + iota) * _RS for bg in range(_NBG)]
             for tl in range(_TCH)]

    def fire_gather(k, p):
        t0 = jnp.minimum(k, _NCH - 1) * _TCH
        for tl in range(_TCH):
            pltpu.async_copy(
                table_hbm.at[idx_v.at[t0 + tl, :]],
                rows_v.at[p, pl.ds(tl * _BPW, _BPW)], sem_g[p])

    def wait_gather(p):
        for _ in range(_TCH):
            pltpu.make_async_copy(
                table_hbm.at[pl.ds(0, _BPW)],
                rows_v.at[p, pl.ds(0, _BPW)], sem_g[p]).wait()

    def fire_write(k, p):
        t0 = k * _TCH
        for tl in range(_TCH):
            pltpu.async_copy(
                out_v.at[p, tl],
                out_hbm.at[t0 + tl, :, wid], sem_w[p])

    def wait_write(p):
        for _ in range(_TCH):
            pltpu.make_async_copy(
                out_v.at[p, 0],
                out_hbm.at[0, :, wid], sem_w[p]).wait()

    def compute(p):
        rows_p = rows_v.at[p]

        @plsc.parallel_loop(0, _TCH * _BPW, step=1, unroll=4)
        def repack(r):
            for q in range(_D // _L):
                rowsP_v[pl.ds(r * _RS + q * _L, _L)] = rows_p[r, pl.ds(q * _L, _L)]

        @plsc.parallel_loop(0, _D, step=1, unroll=4)
        def dloop(d):
            dt = d // 8
            dr = d % 8
            for bg in range(_NBG):
                sl = pl.ds(bg * _L, _L)
                a = acc_v[d, sl]
                for tl in range(_TCH):
                    g = plsc.load_gather(rowsP_v, [gbase[tl][bg] + d])
                    a = a + g * 0.1
                    out_v[p, tl, dt, dr, sl] = a
                acc_v[d, sl] = a

    def chunk(k, p, wait_out):
        wait_gather(p)
        if wait_out:
            wait_write(p)
        compute(p)
        fire_write(k, p)
        fire_gather(k + 2, p)

    fire_gather(0, 0)
    fire_gather(1, 1)
    chunk(0, 0, False)
    chunk(1, 1, False)

    def pair(i, carry):
        chunk(2 * i, 0, True)
        chunk(2 * i + 1, 1, True)
        return carry

    lax.fori_loop(1, _NCH // 2, pair, 0)

    wait_gather(0)
    wait_gather(1)
    wait_write(0)
    wait_write(1)


def kernel(z0, action_seq, act_emb_weight):
    out5 = _dummy_world_rnn(
        act_emb_weight,
        action_seq.astype(jnp.int32).T,
        z0.T,
    )
    return out5.transpose(2, 4, 0, 1, 3).reshape(_B, _T, _D)
